# baseline (speedup 1.0000x reference)
import functools

import jax
import jax.numpy as jnp
from jax import lax
from jax.experimental import pallas as pl
from jax.experimental.pallas import tpu as pltpu
from jax.experimental.pallas import tpu_sc as plsc

RES_INV = 1024.0
BUCKETS = 4194304
HASH_MASK = BUCKETS - 1
P2 = 2654435761
P3 = 805459861
N_PTS = 2097152
D = 2

NUM_CORES = 2
NUM_SUBCORES = 16
LANES = 16
NUM_WORKERS = NUM_CORES * NUM_SUBCORES
PTS_PER_WORKER = N_PTS // NUM_WORKERS
CHUNK = 1024
N_CHUNKS = PTS_PER_WORKER // CHUNK


def _make_sc_kernel():
    mesh = plsc.VectorSubcoreMesh(core_axis_name="c", subcore_axis_name="s")

    scratch = (
        [pltpu.VMEM((3 * CHUNK,), jnp.float32)]
        + [pltpu.VMEM((CHUNK,), jnp.int32) for _ in range(8)]
        + [pltpu.VMEM((CHUNK,), jnp.float32) for _ in range(8)]
        + [pltpu.VMEM((CHUNK, D), jnp.float32) for _ in range(8)]
        + [
            pltpu.VMEM((CHUNK * D,), jnp.float32),
            pltpu.SemaphoreType.DMA,
            pltpu.SemaphoreType.DMA,
        ]
    )

    @functools.partial(
        pl.kernel,
        out_type=jax.ShapeDtypeStruct((N_PTS * D,), jnp.float32),
        mesh=mesh,
        scratch_types=scratch,
        compiler_params=pltpu.CompilerParams(
            needs_layout_passes=False,
            use_tc_tiling_on_sc=False,
        ),
    )
    def sc_kernel(pts_hbm, table_hbm, out_hbm, *refs):
        xyz_v = refs[0]
        idx_v = refs[1:9]
        w_v = refs[9:17]
        rows_v = refs[17:25]
        acc_v, sem_in, sem_g = refs[25:28]

        wid = lax.axis_index("s") * NUM_CORES + lax.axis_index("c")
        lane = lax.iota(jnp.int32, LANES)
        pair = lane >> 1
        feat = lane & 1

        def chunk_body(c, _):
            base = wid * PTS_PER_WORKER + c * CHUNK
            pltpu.async_copy(
                pts_hbm.at[pl.ds(3 * base, 3 * CHUNK)], xyz_v, sem_in
            ).wait()

            def hash_body(j, _):
                o = j * LANES
                p3 = (o + lane) * 3
                qx = plsc.load_gather(xyz_v, [p3]) * RES_INV
                qy = plsc.load_gather(xyz_v, [p3 + 1]) * RES_INV
                qz = plsc.load_gather(xyz_v, [p3 + 2]) * RES_INV
                bx = qx.astype(jnp.int32)
                by = qy.astype(jnp.int32)
                bz = qz.astype(jnp.int32)
                fx = qx - bx.astype(jnp.float32)
                fy = qy - by.astype(jnp.float32)
                fz = qz - bz.astype(jnp.float32)
                hx = (bx.astype(jnp.uint32), bx.astype(jnp.uint32) + jnp.uint32(1))
                hy0 = by.astype(jnp.uint32) * jnp.uint32(P2)
                hy = (hy0, hy0 + jnp.uint32(P2))
                hz0 = bz.astype(jnp.uint32) * jnp.uint32(P3)
                hz = (hz0, hz0 + jnp.uint32(P3))
                wx = (1.0 - fx, fx)
                wy = (1.0 - fy, fy)
                wz = (1.0 - fz, fz)
                for k in range(8):
                    kx, ky, kz = k & 1, (k >> 1) & 1, k >> 2
                    h = (hx[kx] ^ hy[ky] ^ hz[kz]) & jnp.uint32(HASH_MASK)
                    idx_v[k][pl.ds(o, LANES)] = h.astype(jnp.int32)
                    w_v[k][pl.ds(o, LANES)] = wx[kx] * wy[ky] * wz[kz]
                return 0

            lax.fori_loop(0, CHUNK // LANES, hash_body, 0, unroll=2)

            copies = [
                pltpu.async_copy(table_hbm.at[idx_v[k]], rows_v[k], sem_g)
                for k in range(8)
            ]
            for cp in copies:
                cp.wait()

            def blend_body(g, _):
                p = g * 8 + pair
                acc = jnp.zeros((LANES,), jnp.float32)
                for k in range(8):
                    wk = plsc.load_gather(w_v[k], [p])
                    rk = plsc.load_gather(rows_v[k], [p, feat])
                    acc = acc + wk * rk
                acc_v[pl.ds(g * LANES, LANES)] = acc
                return 0

            lax.fori_loop(0, CHUNK // 8, blend_body, 0, unroll=2)

            pltpu.async_copy(
                acc_v, out_hbm.at[pl.ds(base * D, CHUNK * D)], sem_in
            ).wait()
            return 0

        lax.fori_loop(0, N_CHUNKS, chunk_body, 0)

    return sc_kernel


_SC_KERNEL = _make_sc_kernel()


def kernel(pts, voxel_features):
    out_flat = _SC_KERNEL(pts.reshape(-1), voxel_features)
    return out_flat.reshape(N_PTS, D)


# speedup vs baseline: 11.3933x; 11.3933x over previous
import functools

import jax
import jax.numpy as jnp
from jax import lax
from jax.experimental import pallas as pl
from jax.experimental.pallas import tpu as pltpu
from jax.experimental.pallas import tpu_sc as plsc

RES_INV = 1024.0
BUCKETS = 4194304
HASH_MASK = BUCKETS - 1
P2 = 2654435761
P3 = 805459861
N_PTS = 2097152
D = 2

NUM_CORES = 2
NUM_SUBCORES = 16
LANES = 16
NUM_WORKERS = NUM_CORES * NUM_SUBCORES
PTS_PER_WORKER = N_PTS // NUM_WORKERS
CHUNK = 1024
N_CHUNKS = PTS_PER_WORKER // CHUNK


def _make_sc_kernel():
    mesh = plsc.VectorSubcoreMesh(core_axis_name="c", subcore_axis_name="s")

    scratch = (
        [pltpu.VMEM((CHUNK,), jnp.float32) for _ in range(3)]
        + [pltpu.VMEM((CHUNK,), jnp.int32) for _ in range(8)]
        + [pltpu.VMEM((CHUNK,), jnp.float32) for _ in range(8)]
        + [pltpu.VMEM((CHUNK,), jnp.uint32) for _ in range(8)]
        + [
            pltpu.VMEM((CHUNK,), jnp.float32),
            pltpu.VMEM((CHUNK,), jnp.float32),
            pltpu.SemaphoreType.DMA,
            pltpu.SemaphoreType.DMA,
        ]
    )

    @functools.partial(
        pl.kernel,
        out_type=(
            jax.ShapeDtypeStruct((N_PTS,), jnp.float32),
            jax.ShapeDtypeStruct((N_PTS,), jnp.float32),
        ),
        mesh=mesh,
        scratch_types=scratch,
        compiler_params=pltpu.CompilerParams(
            needs_layout_passes=False,
            use_tc_tiling_on_sc=False,
        ),
    )
    def sc_kernel(qx_hbm, qy_hbm, qz_hbm, tbl_hbm, out0_hbm, out1_hbm, *refs):
        q_hbm = (qx_hbm, qy_hbm, qz_hbm)
        q_v = refs[0:3]
        idx_v = refs[3:11]
        w_v = refs[11:19]
        rows_v = refs[19:27]
        acc0_v, acc1_v, sem_in, sem_g = refs[27:31]

        wid = lax.axis_index("s") * NUM_CORES + lax.axis_index("c")

        def chunk_body(c, _):
            base = wid * PTS_PER_WORKER + c * CHUNK
            for a in range(3):
                pltpu.async_copy(
                    q_hbm[a].at[pl.ds(base, CHUNK)], q_v[a], sem_in
                ).wait()

            def hash_body(j, _):
                o = j * LANES
                qx = q_v[0][pl.ds(o, LANES)]
                qy = q_v[1][pl.ds(o, LANES)]
                qz = q_v[2][pl.ds(o, LANES)]
                bx = qx.astype(jnp.int32)
                by = qy.astype(jnp.int32)
                bz = qz.astype(jnp.int32)
                fx = qx - bx.astype(jnp.float32)
                fy = qy - by.astype(jnp.float32)
                fz = qz - bz.astype(jnp.float32)
                hx = (bx.astype(jnp.uint32), bx.astype(jnp.uint32) + jnp.uint32(1))
                hy0 = by.astype(jnp.uint32) * jnp.uint32(P2)
                hy = (hy0, hy0 + jnp.uint32(P2))
                hz0 = bz.astype(jnp.uint32) * jnp.uint32(P3)
                hz = (hz0, hz0 + jnp.uint32(P3))
                wx = (1.0 - fx, fx)
                wy = (1.0 - fy, fy)
                wz = (1.0 - fz, fz)
                for k in range(8):
                    kx, ky, kz = k & 1, (k >> 1) & 1, k >> 2
                    h = (hx[kx] ^ hy[ky] ^ hz[kz]) & jnp.uint32(HASH_MASK)
                    idx_v[k][pl.ds(o, LANES)] = h.astype(jnp.int32)
                    w_v[k][pl.ds(o, LANES)] = wx[kx] * wy[ky] * wz[kz]
                return 0

            lax.fori_loop(0, CHUNK // LANES, hash_body, 0, unroll=2)

            copies = [
                pltpu.async_copy(tbl_hbm.at[idx_v[k]], rows_v[k], sem_g)
                for k in range(8)
            ]
            for cp in copies:
                cp.wait()

            def blend_body(g, _):
                o = g * LANES
                acc0 = jnp.zeros((LANES,), jnp.float32)
                acc1 = jnp.zeros((LANES,), jnp.float32)
                for k in range(8):
                    wk = w_v[k][pl.ds(o, LANES)]
                    u = rows_v[k][pl.ds(o, LANES)]
                    r0 = plsc.bitcast(u & jnp.uint32(0xFFFF0000), jnp.float32)
                    r1 = plsc.bitcast(u << jnp.uint32(16), jnp.float32)
                    acc0 = acc0 + wk * r0
                    acc1 = acc1 + wk * r1
                acc0_v[pl.ds(o, LANES)] = acc0
                acc1_v[pl.ds(o, LANES)] = acc1
                return 0

            lax.fori_loop(0, CHUNK // LANES, blend_body, 0, unroll=2)

            pltpu.async_copy(acc0_v, out0_hbm.at[pl.ds(base, CHUNK)], sem_in).wait()
            pltpu.async_copy(acc1_v, out1_hbm.at[pl.ds(base, CHUNK)], sem_in).wait()
            return 0

        lax.fori_loop(0, N_CHUNKS, chunk_body, 0)

    return sc_kernel


_SC_KERNEL = _make_sc_kernel()


def kernel(pts, voxel_features):
    qx = pts[:, 0] * RES_INV
    qy = pts[:, 1] * RES_INV
    qz = pts[:, 2] * RES_INV
    bits = jax.lax.bitcast_convert_type(voxel_features, jnp.uint32)
    r = jnp.uint32(0x8000)
    packed = ((bits[:, 0] + r) & jnp.uint32(0xFFFF0000)) | ((bits[:, 1] + r) >> 16)
    out0, out1 = _SC_KERNEL(qx, qy, qz, packed)
    return jnp.stack([out0, out1], axis=1)


# speedup vs baseline: 14.8260x; 1.3013x over previous
import functools

import jax
import jax.numpy as jnp
from jax import lax
from jax.experimental import pallas as pl
from jax.experimental.pallas import tpu as pltpu
from jax.experimental.pallas import tpu_sc as plsc

RES_INV = 1024.0
BUCKETS = 4194304
HASH_MASK = BUCKETS - 1
P2 = 2654435761
P3 = 805459861
N_PTS = 2097152
D = 2

NUM_CORES = 2
NUM_SUBCORES = 16
LANES = 16
NUM_WORKERS = NUM_CORES * NUM_SUBCORES
PTS_PER_WORKER = N_PTS // NUM_WORKERS
CHUNK = 1024
N_CHUNKS = PTS_PER_WORKER // CHUNK
assert N_CHUNKS % 2 == 0 and N_CHUNKS >= 4


def _make_sc_kernel():
    mesh = plsc.VectorSubcoreMesh(core_axis_name="c", subcore_axis_name="s")

    def one_set():
        return (
            [pltpu.VMEM((CHUNK,), jnp.float32) for _ in range(3)]
            + [pltpu.VMEM((CHUNK,), jnp.int32) for _ in range(8)]
            + [pltpu.VMEM((CHUNK,), jnp.float32) for _ in range(8)]
            + [pltpu.VMEM((CHUNK,), jnp.uint32) for _ in range(8)]
            + [pltpu.VMEM((CHUNK,), jnp.float32) for _ in range(2)]
            + [pltpu.SemaphoreType.DMA for _ in range(3)]
        )

    scratch = one_set() + one_set()
    SET_LEN = len(scratch) // 2

    @functools.partial(
        pl.kernel,
        out_type=(
            jax.ShapeDtypeStruct((N_PTS,), jnp.float32),
            jax.ShapeDtypeStruct((N_PTS,), jnp.float32),
        ),
        mesh=mesh,
        scratch_types=scratch,
        compiler_params=pltpu.CompilerParams(
            needs_layout_passes=False,
            use_tc_tiling_on_sc=False,
        ),
    )
    def sc_kernel(qx_hbm, qy_hbm, qz_hbm, tbl_hbm, out0_hbm, out1_hbm, *refs):
        q_hbm = (qx_hbm, qy_hbm, qz_hbm)
        out_hbm = (out0_hbm, out1_hbm)
        sets = (refs[:SET_LEN], refs[SET_LEN:])

        def parts(S):
            r = sets[S]
            return r[0:3], r[3:11], r[11:19], r[19:27], r[27:29], r[29:32]

        wid = lax.axis_index("s") * NUM_CORES + lax.axis_index("c")

        def base_of(c):
            return wid * PTS_PER_WORKER + c * CHUNK

        def fire_L(S, c):
            q_v, _, _, _, _, (semL, _, _) = parts(S)
            b = base_of(c)
            for a in range(3):
                pltpu.async_copy(q_hbm[a].at[pl.ds(b, CHUNK)], q_v[a], semL)

        def wait_L(S, c):
            q_v, _, _, _, _, (semL, _, _) = parts(S)
            b = base_of(c)
            for a in range(3):
                pltpu.make_async_copy(
                    q_hbm[a].at[pl.ds(b, CHUNK)], q_v[a], semL
                ).wait()

        def fire_G(S):
            _, idx_v, _, rows_v, _, (_, semG, _) = parts(S)
            for k in range(8):
                pltpu.async_copy(tbl_hbm.at[idx_v[k]], rows_v[k], semG)

        def wait_G(S):
            _, idx_v, _, rows_v, _, (_, semG, _) = parts(S)
            for k in range(8):
                pltpu.make_async_copy(tbl_hbm.at[idx_v[k]], rows_v[k], semG).wait()

        def fire_O(S, c):
            _, _, _, _, acc_v, (_, _, semO) = parts(S)
            b = base_of(c)
            for f in range(2):
                pltpu.async_copy(acc_v[f], out_hbm[f].at[pl.ds(b, CHUNK)], semO)

        def wait_O(S, c):
            _, _, _, _, acc_v, (_, _, semO) = parts(S)
            b = base_of(c)
            for f in range(2):
                pltpu.make_async_copy(
                    acc_v[f], out_hbm[f].at[pl.ds(b, CHUNK)], semO
                ).wait()

        def hash_pass(S):
            q_v, idx_v, w_v, _, _, _ = parts(S)

            def hash_body(j, _):
                o = j * LANES
                qx = q_v[0][pl.ds(o, LANES)]
                qy = q_v[1][pl.ds(o, LANES)]
                qz = q_v[2][pl.ds(o, LANES)]
                bx = qx.astype(jnp.int32)
                by = qy.astype(jnp.int32)
                bz = qz.astype(jnp.int32)
                fx = qx - bx.astype(jnp.float32)
                fy = qy - by.astype(jnp.float32)
                fz = qz - bz.astype(jnp.float32)
                hx = (bx.astype(jnp.uint32), bx.astype(jnp.uint32) + jnp.uint32(1))
                hy0 = by.astype(jnp.uint32) * jnp.uint32(P2)
                hy = (hy0, hy0 + jnp.uint32(P2))
                hz0 = bz.astype(jnp.uint32) * jnp.uint32(P3)
                hz = (hz0, hz0 + jnp.uint32(P3))
                wx = (1.0 - fx, fx)
                wy = (1.0 - fy, fy)
                wz = (1.0 - fz, fz)
                for k in range(8):
                    kx, ky, kz = k & 1, (k >> 1) & 1, k >> 2
                    h = (hx[kx] ^ hy[ky] ^ hz[kz]) & jnp.uint32(HASH_MASK)
                    idx_v[k][pl.ds(o, LANES)] = h.astype(jnp.int32)
                    w_v[k][pl.ds(o, LANES)] = wx[kx] * wy[ky] * wz[kz]
                return 0

            lax.fori_loop(0, CHUNK // LANES, hash_body, 0, unroll=2)

        def blend_pass(S):
            _, _, w_v, rows_v, acc_v, _ = parts(S)

            def blend_body(g, _):
                o = g * LANES
                acc0 = jnp.zeros((LANES,), jnp.float32)
                acc1 = jnp.zeros((LANES,), jnp.float32)
                for k in range(8):
                    wk = w_v[k][pl.ds(o, LANES)]
                    u = rows_v[k][pl.ds(o, LANES)]
                    r0 = plsc.bitcast(u & jnp.uint32(0xFFFF0000), jnp.float32)
                    r1 = plsc.bitcast(u << jnp.uint32(16), jnp.float32)
                    acc0 = acc0 + wk * r0
                    acc1 = acc1 + wk * r1
                acc_v[0][pl.ds(o, LANES)] = acc0
                acc_v[1][pl.ds(o, LANES)] = acc1
                return 0

            lax.fori_loop(0, CHUNK // LANES, blend_body, 0, unroll=2)

        fire_L(0, 0)
        wait_L(0, 0)
        hash_pass(0)
        fire_G(0)
        fire_L(1, 1)

        def pair_body(t, _):
            c0 = 2 * t
            wait_L(1, c0 + 1)
            hash_pass(1)
            fire_G(1)
            fire_L(0, c0 + 2)
            wait_G(0)

            @pl.when(t > 0)
            def _():
                wait_O(0, c0 - 2)

            blend_pass(0)
            fire_O(0, c0)
            wait_L(0, c0 + 2)
            hash_pass(0)
            fire_G(0)
            fire_L(1, c0 + 3)
            wait_G(1)

            @pl.when(t > 0)
            def _():
                wait_O(1, c0 - 1)

            blend_pass(1)
            fire_O(1, c0 + 1)
            return 0

        lax.fori_loop(0, (N_CHUNKS - 2) // 2, pair_body, 0)

        cl = N_CHUNKS - 2
        wait_L(1, cl + 1)
        hash_pass(1)
        fire_G(1)
        wait_G(0)
        wait_O(0, cl - 2)
        blend_pass(0)
        fire_O(0, cl)
        wait_G(1)
        wait_O(1, cl - 1)
        blend_pass(1)
        fire_O(1, cl + 1)
        wait_O(0, cl)
        wait_O(1, cl + 1)

    return sc_kernel


_SC_KERNEL = _make_sc_kernel()


def kernel(pts, voxel_features):
    qx = pts[:, 0] * RES_INV
    qy = pts[:, 1] * RES_INV
    qz = pts[:, 2] * RES_INV
    bits = jax.lax.bitcast_convert_type(voxel_features, jnp.uint32)
    r = jnp.uint32(0x8000)
    packed = ((bits[:, 0] + r) & jnp.uint32(0xFFFF0000)) | ((bits[:, 1] + r) >> 16)
    out0, out1 = _SC_KERNEL(qx, qy, qz, packed)
    return jnp.stack([out0, out1], axis=1)


# speedup vs baseline: 14.8616x; 1.0024x over previous
import functools

import jax
import jax.numpy as jnp
from jax import lax
from jax.experimental import pallas as pl
from jax.experimental.pallas import tpu as pltpu
from jax.experimental.pallas import tpu_sc as plsc

RES_INV = 1024.0
BUCKETS = 4194304
HASH_MASK = BUCKETS - 1
P2 = 2654435761
P3 = 805459861
N_PTS = 2097152
D = 2

NUM_CORES = 2
NUM_SUBCORES = 16
LANES = 16
NUM_WORKERS = NUM_CORES * NUM_SUBCORES
PTS_PER_WORKER = N_PTS // NUM_WORKERS
CHUNK = 1024
N_CHUNKS = PTS_PER_WORKER // CHUNK
assert N_CHUNKS % 2 == 0 and N_CHUNKS >= 4


def _make_sc_kernel():
    mesh = plsc.VectorSubcoreMesh(core_axis_name="c", subcore_axis_name="s")

    def one_set():
        return (
            [pltpu.VMEM((CHUNK,), jnp.float32) for _ in range(3)]
            + [pltpu.VMEM((8 * CHUNK,), jnp.int32)]
            + [pltpu.VMEM((CHUNK,), jnp.float32) for _ in range(8)]
            + [pltpu.VMEM((8 * CHUNK,), jnp.uint32)]
            + [pltpu.VMEM((CHUNK,), jnp.float32) for _ in range(2)]
            + [pltpu.SemaphoreType.DMA for _ in range(3)]
        )

    scratch = one_set() + one_set()
    SET_LEN = len(scratch) // 2

    @functools.partial(
        pl.kernel,
        out_type=(
            jax.ShapeDtypeStruct((N_PTS,), jnp.float32),
            jax.ShapeDtypeStruct((N_PTS,), jnp.float32),
        ),
        mesh=mesh,
        scratch_types=scratch,
        compiler_params=pltpu.CompilerParams(
            needs_layout_passes=False,
            use_tc_tiling_on_sc=False,
        ),
    )
    def sc_kernel(qx_hbm, qy_hbm, qz_hbm, tbl_hbm, out0_hbm, out1_hbm, *refs):
        q_hbm = (qx_hbm, qy_hbm, qz_hbm)
        out_hbm = (out0_hbm, out1_hbm)
        sets = (refs[:SET_LEN], refs[SET_LEN:])

        def parts(S):
            r = sets[S]
            return r[0:3], r[3], r[4:12], r[12], r[13:15], r[15:18]

        wid = lax.axis_index("s") * NUM_CORES + lax.axis_index("c")

        def base_of(c):
            return wid * PTS_PER_WORKER + c * CHUNK

        def fire_L(S, c):
            q_v, _, _, _, _, (semL, _, _) = parts(S)
            b = base_of(c)
            for a in range(3):
                pltpu.async_copy(q_hbm[a].at[pl.ds(b, CHUNK)], q_v[a], semL)

        def wait_L(S, c):
            q_v, _, _, _, _, (semL, _, _) = parts(S)
            b = base_of(c)
            for a in range(3):
                pltpu.make_async_copy(
                    q_hbm[a].at[pl.ds(b, CHUNK)], q_v[a], semL
                ).wait()

        def fire_G(S):
            _, idx_v, _, rows_v, _, (_, semG, _) = parts(S)
            pltpu.async_copy(tbl_hbm.at[idx_v], rows_v, semG)

        def wait_G(S):
            _, idx_v, _, rows_v, _, (_, semG, _) = parts(S)
            pltpu.make_async_copy(tbl_hbm.at[idx_v], rows_v, semG).wait()

        def fire_O(S, c):
            _, _, _, _, acc_v, (_, _, semO) = parts(S)
            b = base_of(c)
            for f in range(2):
                pltpu.async_copy(acc_v[f], out_hbm[f].at[pl.ds(b, CHUNK)], semO)

        def wait_O(S, c):
            _, _, _, _, acc_v, (_, _, semO) = parts(S)
            b = base_of(c)
            for f in range(2):
                pltpu.make_async_copy(
                    acc_v[f], out_hbm[f].at[pl.ds(b, CHUNK)], semO
                ).wait()

        def hash_pass(S):
            q_v, idx_v, w_v, _, _, _ = parts(S)

            def hash_body(j, _):
                o = j * LANES
                qx = q_v[0][pl.ds(o, LANES)]
                qy = q_v[1][pl.ds(o, LANES)]
                qz = q_v[2][pl.ds(o, LANES)]
                bx = qx.astype(jnp.int32)
                by = qy.astype(jnp.int32)
                bz = qz.astype(jnp.int32)
                fx = qx - bx.astype(jnp.float32)
                fy = qy - by.astype(jnp.float32)
                fz = qz - bz.astype(jnp.float32)
                hx = (bx.astype(jnp.uint32), bx.astype(jnp.uint32) + jnp.uint32(1))
                hy0 = by.astype(jnp.uint32) * jnp.uint32(P2)
                hy = (hy0, hy0 + jnp.uint32(P2))
                hz0 = bz.astype(jnp.uint32) * jnp.uint32(P3)
                hz = (hz0, hz0 + jnp.uint32(P3))
                wx = (1.0 - fx, fx)
                wy = (1.0 - fy, fy)
                wz = (1.0 - fz, fz)
                for k in range(8):
                    kx, ky, kz = k & 1, (k >> 1) & 1, k >> 2
                    h = (hx[kx] ^ hy[ky] ^ hz[kz]) & jnp.uint32(HASH_MASK)
                    idx_v[pl.ds(k * CHUNK + o, LANES)] = h.astype(jnp.int32)
                    w_v[k][pl.ds(o, LANES)] = wx[kx] * wy[ky] * wz[kz]
                return 0

            lax.fori_loop(0, CHUNK // LANES, hash_body, 0, unroll=2)

        def blend_pass(S):
            _, _, w_v, rows_v, acc_v, _ = parts(S)

            def blend_body(g, _):
                o = g * LANES
                acc0 = jnp.zeros((LANES,), jnp.float32)
                acc1 = jnp.zeros((LANES,), jnp.float32)
                for k in range(8):
                    wk = w_v[k][pl.ds(o, LANES)]
                    u = rows_v[pl.ds(k * CHUNK + o, LANES)]
                    r0 = plsc.bitcast(u & jnp.uint32(0xFFFF0000), jnp.float32)
                    r1 = plsc.bitcast(u << jnp.uint32(16), jnp.float32)
                    acc0 = acc0 + wk * r0
                    acc1 = acc1 + wk * r1
                acc_v[0][pl.ds(o, LANES)] = acc0
                acc_v[1][pl.ds(o, LANES)] = acc1
                return 0

            lax.fori_loop(0, CHUNK // LANES, blend_body, 0, unroll=2)

        fire_L(0, 0)
        wait_L(0, 0)
        hash_pass(0)
        fire_G(0)
        fire_L(1, 1)

        def pair_body(t, _):
            c0 = 2 * t
            wait_L(1, c0 + 1)
            hash_pass(1)
            fire_G(1)
            fire_L(0, c0 + 2)
            wait_G(0)

            @pl.when(t > 0)
            def _():
                wait_O(0, c0 - 2)

            blend_pass(0)
            fire_O(0, c0)
            wait_L(0, c0 + 2)
            hash_pass(0)
            fire_G(0)
            fire_L(1, c0 + 3)
            wait_G(1)

            @pl.when(t > 0)
            def _():
                wait_O(1, c0 - 1)

            blend_pass(1)
            fire_O(1, c0 + 1)
            return 0

        lax.fori_loop(0, (N_CHUNKS - 2) // 2, pair_body, 0)

        cl = N_CHUNKS - 2
        wait_L(1, cl + 1)
        hash_pass(1)
        fire_G(1)
        wait_G(0)
        wait_O(0, cl - 2)
        blend_pass(0)
        fire_O(0, cl)
        wait_G(1)
        wait_O(1, cl - 1)
        blend_pass(1)
        fire_O(1, cl + 1)
        wait_O(0, cl)
        wait_O(1, cl + 1)

    return sc_kernel


_SC_KERNEL = _make_sc_kernel()


def kernel(pts, voxel_features):
    qx = pts[:, 0] * RES_INV
    qy = pts[:, 1] * RES_INV
    qz = pts[:, 2] * RES_INV
    bits = jax.lax.bitcast_convert_type(voxel_features, jnp.uint32)
    r = jnp.uint32(0x8000)
    packed = ((bits[:, 0] + r) & jnp.uint32(0xFFFF0000)) | ((bits[:, 1] + r) >> 16)
    out0, out1 = _SC_KERNEL(qx, qy, qz, packed)
    return jnp.stack([out0, out1], axis=1)
